# Initial kernel scaffold; baseline (speedup 1.0000x reference)
import jax
import jax.numpy as jnp
from jax.experimental import pallas as pl


def kernel(x, W_router):
    raise NotImplementedError("write your pallas kernel here")


# speedup vs baseline: 1.1392x; 1.1392x over previous
import functools

import jax
import jax.numpy as jnp
from jax.experimental import pallas as pl
from jax.experimental.pallas import tpu as pltpu

HIDDEN_DIM = 768
N_EXPERTS = 64
TOPK = 8
TOKEN_BLOCK = 1024


def _router_block(x_ref, w_ref, wts_ref, idx_ref):
    xb = x_ref[...]
    wb = w_ref[...]
    logits = jax.lax.dot_general(
        xb, wb, (((1,), (1,)), ((), ())), preferred_element_type=jnp.float32
    )
    t = logits.shape[0]
    eiota = jax.lax.broadcasted_iota(jnp.int32, (t, N_EXPERTS), 1)
    neg_inf = jnp.float32(-jnp.inf)

    cur = logits
    vals = []
    idxs = []
    for _ in range(TOPK):
        m = jnp.max(cur, axis=1, keepdims=True)
        hit = cur == m
        i = jnp.min(jnp.where(hit, eiota, N_EXPERTS), axis=1, keepdims=True)
        vals.append(m)
        idxs.append(i)
        cur = jnp.where(eiota == i, neg_inf, cur)

    v = jnp.concatenate(vals, axis=1)
    e = jnp.exp(v - v[:, 0:1])
    wts_ref[...] = e / jnp.sum(e, axis=1, keepdims=True)
    idx_ref[...] = jnp.concatenate(idxs, axis=1)


@functools.partial(jax.jit, static_argnames=())
def kernel(x, W_router):
    n_tokens = x.shape[0] * x.shape[1]
    x_flat = x.reshape(n_tokens, HIDDEN_DIM)
    grid = (n_tokens // TOKEN_BLOCK,)
    wts, idx = pl.pallas_call(
        _router_block,
        grid=grid,
        in_specs=[
            pl.BlockSpec((TOKEN_BLOCK, HIDDEN_DIM), lambda i: (i, 0)),
            pl.BlockSpec((N_EXPERTS, HIDDEN_DIM), lambda i: (0, 0)),
        ],
        out_specs=[
            pl.BlockSpec((TOKEN_BLOCK, TOPK), lambda i: (i, 0)),
            pl.BlockSpec((TOKEN_BLOCK, TOPK), lambda i: (i, 0)),
        ],
        out_shape=[
            jax.ShapeDtypeStruct((n_tokens, TOPK), jnp.float32),
            jax.ShapeDtypeStruct((n_tokens, TOPK), jnp.int32),
        ],
        compiler_params=pltpu.CompilerParams(
            dimension_semantics=("arbitrary",),
        ),
    )(x_flat, W_router)
    return wts, idx


# speedup vs baseline: 1.5573x; 1.3669x over previous
import functools

import jax
import jax.numpy as jnp
from jax.experimental import pallas as pl
from jax.experimental.pallas import tpu as pltpu

HIDDEN_DIM = 768
N_EXPERTS = 64
TOPK = 8
TOKEN_BLOCK = 1024


def _router_block(x_ref, w_ref, wts_ref, idx_ref):
    xb = x_ref[...]
    wb = w_ref[...]
    logits = jax.lax.dot_general(
        xb, wb, (((1,), (1,)), ((), ())), preferred_element_type=jnp.float32
    )
    t = logits.shape[0]
    fiota = jax.lax.broadcasted_iota(jnp.int32, (t, N_EXPERTS), 1).astype(
        jnp.float32
    )
    neg_inf = jnp.float32(-jnp.inf)

    cur = logits
    vals = []
    idxs = []
    for _ in range(TOPK):
        m = jnp.max(cur, axis=1, keepdims=True)
        i = jnp.min(
            jnp.where(cur == m, fiota, jnp.float32(N_EXPERTS)),
            axis=1,
            keepdims=True,
        )
        vals.append(m)
        idxs.append(i)
        cur = jnp.where(fiota == i, neg_inf, cur)

    v = jnp.concatenate(vals, axis=1)
    e = jnp.exp(v - v[:, 0:1])
    wts_ref[...] = e / jnp.sum(e, axis=1, keepdims=True)
    idx_ref[...] = jnp.concatenate(idxs, axis=1).astype(jnp.int32)


@functools.partial(jax.jit, static_argnames=())
def kernel(x, W_router):
    n_tokens = x.shape[0] * x.shape[1]
    x_flat = x.reshape(n_tokens, HIDDEN_DIM)
    grid = (n_tokens // TOKEN_BLOCK,)
    wts, idx = pl.pallas_call(
        _router_block,
        grid=grid,
        in_specs=[
            pl.BlockSpec((TOKEN_BLOCK, HIDDEN_DIM), lambda i: (i, 0)),
            pl.BlockSpec((N_EXPERTS, HIDDEN_DIM), lambda i: (0, 0)),
        ],
        out_specs=[
            pl.BlockSpec((TOKEN_BLOCK, TOPK), lambda i: (i, 0)),
            pl.BlockSpec((TOKEN_BLOCK, TOPK), lambda i: (i, 0)),
        ],
        out_shape=[
            jax.ShapeDtypeStruct((n_tokens, TOPK), jnp.float32),
            jax.ShapeDtypeStruct((n_tokens, TOPK), jnp.int32),
        ],
        compiler_params=pltpu.CompilerParams(
            dimension_semantics=("arbitrary",),
        ),
    )(x_flat, W_router)
    return wts, idx


# speedup vs baseline: 3.4042x; 2.1860x over previous
import jax
import jax.numpy as jnp
from jax.experimental import pallas as pl
from jax.experimental.pallas import tpu as pltpu

HIDDEN_DIM = 768
N_EXPERTS = 64
TOPK = 8
TOKEN_BLOCK = 1024


def _router_block(x_ref, w_ref, wts_ref, idx_ref):
    xb = x_ref[...]
    wb = w_ref[...]
    logits = jax.lax.dot_general(
        wb, xb, (((1,), (1,)), ((), ())), preferred_element_type=jnp.float32
    )
    t = logits.shape[1]
    fiota = jax.lax.broadcasted_iota(jnp.int32, (N_EXPERTS, t), 0).astype(
        jnp.float32
    )
    neg_inf = jnp.float32(-jnp.inf)

    cur = logits
    vals = []
    idxs = []
    for k in range(TOPK):
        m = jnp.max(cur, axis=0, keepdims=True)
        i = jnp.min(
            jnp.where(cur == m, fiota, jnp.float32(N_EXPERTS)),
            axis=0,
            keepdims=True,
        )
        vals.append(m)
        idxs.append(i)
        if k + 1 < TOPK:
            cur = jnp.where(fiota == i, neg_inf, cur)

    es = [jnp.ones_like(vals[0])]
    es += [jnp.exp(v - vals[0]) for v in vals[1:]]
    s = es[0]
    for e in es[1:]:
        s = s + e
    r = jnp.float32(1.0) / s
    for k in range(TOPK):
        wts_ref[k : k + 1, :] = es[k] * r
        idx_ref[k : k + 1, :] = idxs[k].astype(jnp.int32)


def kernel(x, W_router):
    n_tokens = x.shape[0] * x.shape[1]
    x_flat = x.reshape(n_tokens, HIDDEN_DIM)
    grid = (n_tokens // TOKEN_BLOCK,)
    wts_t, idx_t = pl.pallas_call(
        _router_block,
        grid=grid,
        in_specs=[
            pl.BlockSpec((TOKEN_BLOCK, HIDDEN_DIM), lambda i: (i, 0)),
            pl.BlockSpec((N_EXPERTS, HIDDEN_DIM), lambda i: (0, 0)),
        ],
        out_specs=[
            pl.BlockSpec((TOPK, TOKEN_BLOCK), lambda i: (0, i)),
            pl.BlockSpec((TOPK, TOKEN_BLOCK), lambda i: (0, i)),
        ],
        out_shape=[
            jax.ShapeDtypeStruct((TOPK, n_tokens), jnp.float32),
            jax.ShapeDtypeStruct((TOPK, n_tokens), jnp.int32),
        ],
        compiler_params=pltpu.CompilerParams(
            dimension_semantics=("arbitrary",),
        ),
    )(x_flat, W_router)
    return wts_t.T, idx_t.T


# speedup vs baseline: 4.1921x; 1.2314x over previous
import jax
import jax.numpy as jnp
from jax.experimental import pallas as pl
from jax.experimental.pallas import tpu as pltpu

HIDDEN_DIM = 768
N_EXPERTS = 64
TOPK = 8
TOKEN_BLOCK = 2048


def _router_block(x_ref, w_ref, wts_ref, idx_ref):
    xb = x_ref[...]
    wb = w_ref[...]
    logits = jax.lax.dot_general(
        wb, xb, (((1,), (1,)), ((), ())), preferred_element_type=jnp.float32
    )
    t = logits.shape[1]
    fiota = jax.lax.broadcasted_iota(jnp.int32, (N_EXPERTS, t), 0).astype(
        jnp.float32
    )
    neg_inf = jnp.float32(-jnp.inf)

    cur = logits
    vals = []
    idxs = []
    for k in range(TOPK):
        m = jnp.max(cur, axis=0, keepdims=True)
        i = jnp.min(
            jnp.where(cur == m, fiota, jnp.float32(N_EXPERTS)),
            axis=0,
            keepdims=True,
        )
        vals.append(m)
        idxs.append(i)
        if k + 1 < TOPK:
            cur = jnp.where(fiota == i, neg_inf, cur)

    es = [jnp.ones_like(vals[0])]
    es += [jnp.exp(v - vals[0]) for v in vals[1:]]
    s = es[0]
    for e in es[1:]:
        s = s + e
    r = jnp.float32(1.0) / s
    for k in range(TOPK):
        wts_ref[k : k + 1, :] = es[k] * r
        idx_ref[k : k + 1, :] = idxs[k].astype(jnp.int32)


def kernel(x, W_router):
    n_tokens = x.shape[0] * x.shape[1]
    x_flat = x.reshape(n_tokens, HIDDEN_DIM)
    grid = (n_tokens // TOKEN_BLOCK,)
    wts_t, idx_t = pl.pallas_call(
        _router_block,
        grid=grid,
        in_specs=[
            pl.BlockSpec((TOKEN_BLOCK, HIDDEN_DIM), lambda i: (i, 0)),
            pl.BlockSpec((N_EXPERTS, HIDDEN_DIM), lambda i: (0, 0)),
        ],
        out_specs=[
            pl.BlockSpec((TOPK, TOKEN_BLOCK), lambda i: (0, i)),
            pl.BlockSpec((TOPK, TOKEN_BLOCK), lambda i: (0, i)),
        ],
        out_shape=[
            jax.ShapeDtypeStruct((TOPK, n_tokens), jnp.float32),
            jax.ShapeDtypeStruct((TOPK, n_tokens), jnp.int32),
        ],
        compiler_params=pltpu.CompilerParams(
            dimension_semantics=("arbitrary",),
        ),
    )(x_flat, W_router)
    return wts_t.T, idx_t.T


# speedup vs baseline: 4.6480x; 1.1088x over previous
import jax
import jax.numpy as jnp
from jax.experimental import pallas as pl
from jax.experimental.pallas import tpu as pltpu

HIDDEN_DIM = 768
N_EXPERTS = 64
TOPK = 8
TOKEN_BLOCK = 4096


def _router_block(x_ref, w_ref, wts_ref, idx_ref):
    xb = x_ref[...]
    wb = w_ref[...]
    logits = jax.lax.dot_general(
        wb, xb, (((1,), (1,)), ((), ())), preferred_element_type=jnp.float32
    )
    t = logits.shape[1]
    fiota = jax.lax.broadcasted_iota(jnp.int32, (N_EXPERTS, t), 0).astype(
        jnp.float32
    )
    neg_inf = jnp.float32(-jnp.inf)

    cur = logits
    vals = []
    idxs = []
    for k in range(TOPK):
        m = jnp.max(cur, axis=0, keepdims=True)
        i = jnp.min(
            jnp.where(cur == m, fiota, jnp.float32(N_EXPERTS)),
            axis=0,
            keepdims=True,
        )
        vals.append(m)
        idxs.append(i)
        if k + 1 < TOPK:
            cur = jnp.where(fiota == i, neg_inf, cur)

    es = [jnp.ones_like(vals[0])]
    es += [jnp.exp(v - vals[0]) for v in vals[1:]]
    s = es[0]
    for e in es[1:]:
        s = s + e
    r = jnp.float32(1.0) / s
    for k in range(TOPK):
        wts_ref[k : k + 1, :] = es[k] * r
        idx_ref[k : k + 1, :] = idxs[k].astype(jnp.int32)


def kernel(x, W_router):
    n_tokens = x.shape[0] * x.shape[1]
    x_flat = x.reshape(n_tokens, HIDDEN_DIM)
    grid = (n_tokens // TOKEN_BLOCK,)
    wts_t, idx_t = pl.pallas_call(
        _router_block,
        grid=grid,
        in_specs=[
            pl.BlockSpec((TOKEN_BLOCK, HIDDEN_DIM), lambda i: (i, 0)),
            pl.BlockSpec((N_EXPERTS, HIDDEN_DIM), lambda i: (0, 0)),
        ],
        out_specs=[
            pl.BlockSpec((TOPK, TOKEN_BLOCK), lambda i: (0, i)),
            pl.BlockSpec((TOPK, TOKEN_BLOCK), lambda i: (0, i)),
        ],
        out_shape=[
            jax.ShapeDtypeStruct((TOPK, n_tokens), jnp.float32),
            jax.ShapeDtypeStruct((TOPK, n_tokens), jnp.int32),
        ],
        compiler_params=pltpu.CompilerParams(
            dimension_semantics=("arbitrary",),
        ),
    )(x_flat, W_router)
    return wts_t.T, idx_t.T
